# baseline (speedup 1.0000x reference)
import functools

import jax
import jax.numpy as jnp
from jax import lax
from jax.experimental import pallas as pl
from jax.experimental.pallas import tpu as pltpu
from jax.experimental.pallas import tpu_sc as plsc

N = 10000
NP = 10240
D = 128
E = 320000
NC, NS, L = 2, 16, 16
NT = NC * NS
CH = 64
C = 160
ET = C * CH
EP = ET * NT
R2 = EP // CH
SL = NP // NS
BLK = 512
NB = NP // BLK


def _deg_body(src_hbm, dst_hbm, ones_hbm, zeros_hbm, out_hbm,
              src_v, dst_v, ones_v, degs_sh, degd_sh, sem_s, sem_d):
    c = lax.axis_index("c")
    s = lax.axis_index("s")
    w = c * NS + s
    pltpu.sync_copy(src_hbm.at[pl.ds(w * C, C)], src_v)
    pltpu.sync_copy(dst_hbm.at[pl.ds(w * C, C)], dst_v)
    pltpu.sync_copy(ones_hbm, ones_v)
    pltpu.sync_copy(zeros_hbm, degs_sh.at[pl.ds(s * SL, SL)])
    pltpu.sync_copy(zeros_hbm, degd_sh.at[pl.ds(s * SL, SL)])
    plsc.subcore_barrier()

    def body(jj, carry):
        for u in range(4):
            j = jj * 4 + u
            pltpu.async_copy(ones_v, degs_sh.at[src_v.at[j]], sem_s,
                             add=True)
            pltpu.async_copy(ones_v, degd_sh.at[dst_v.at[j]], sem_d,
                             add=True)
        for u in range(4):
            j = jj * 4 + u
            pltpu.make_async_copy(ones_v, degs_sh.at[src_v.at[j]],
                                  sem_s).wait()
            pltpu.make_async_copy(ones_v, degd_sh.at[dst_v.at[j]],
                                  sem_d).wait()
        return carry

    lax.fori_loop(0, C // 4, body, 0)
    plsc.subcore_barrier()
    sl = pl.ds(s * SL, SL)
    pltpu.sync_copy(degs_sh.at[sl], out_hbm.at[c, 0, sl])
    pltpu.sync_copy(degd_sh.at[sl], out_hbm.at[c, 1, sl])


CHS = 128
CT = EP // (CHS * NT)
S0 = 108
S1 = 2 * CT - S0
RS = EP // CHS
RI = 4
RR = 2


def _spmm_run(h_hbm, src_hbm, dst_hbm, acc_sh,
              sidx, didx, rows, sisems, disems, gsems, ssems, S, base):
    for b in range(RI):
        pltpu.async_copy(src_hbm.at[base + b], sidx.at[b], sisems.at[b])
        pltpu.async_copy(dst_hbm.at[base + b], didx.at[b], disems.at[b])

    def body(jj, carry):
        for b in range(RI):
            j = jj * RI + b
            r = b % RR
            b2 = (b + 2) % RI
            pltpu.make_async_copy(src_hbm.at[base + j], sidx.at[b],
                                  sisems.at[b]).wait()
            pltpu.make_async_copy(dst_hbm.at[base + j], didx.at[b],
                                  disems.at[b]).wait()

            def drain_and_prefetch():
                pltpu.make_async_copy(rows.at[r], acc_sh.at[didx.at[b2]],
                                      ssems.at[r]).wait()
                pltpu.async_copy(src_hbm.at[base + j + 2], sidx.at[b2],
                                 sisems.at[b2])
                pltpu.async_copy(dst_hbm.at[base + j + 2], didx.at[b2],
                                 disems.at[b2])

            if b >= 2:
                if b >= RI - 2:
                    @pl.when(jj < S // RI - 1)
                    def _():
                        drain_and_prefetch()

                    @pl.when(jj == S // RI - 1)
                    def _():
                        pltpu.make_async_copy(
                            rows.at[r], acc_sh.at[didx.at[b2]], ssems.at[r]
                        ).wait()
                else:
                    drain_and_prefetch()
            else:
                @pl.when(jj > 0)
                def _():
                    drain_and_prefetch()

            pltpu.async_copy(h_hbm.at[sidx.at[b]], rows.at[r], gsems.at[r])
            pltpu.make_async_copy(h_hbm.at[sidx.at[b]], rows.at[r],
                                  gsems.at[r]).wait()
            pltpu.async_copy(rows.at[r], acc_sh.at[didx.at[b]],
                             ssems.at[r], add=True)
        return carry

    lax.fori_loop(0, S // RI, body, 0)
    for r in range(RR):
        pltpu.make_async_copy(rows.at[r], acc_sh.at[didx.at[r]],
                              ssems.at[r]).wait()


def _spmm_body(h_hbm, src_hbm, dst_hbm, zeros_hbm, out_hbm,
               sidx, didx, rows, acc_sh, sisems, disems, gsems, ssems):
    c = lax.axis_index("c")
    s = lax.axis_index("s")
    pltpu.sync_copy(zeros_hbm, acc_sh.at[pl.ds(s * SL, SL)])
    plsc.subcore_barrier()

    @pl.when(c == 0)
    def _():
        _spmm_run(h_hbm, src_hbm, dst_hbm, acc_sh, sidx, didx, rows,
                  sisems, disems, gsems, ssems, S0, s * S0)

    @pl.when(c == 1)
    def _():
        _spmm_run(h_hbm, src_hbm, dst_hbm, acc_sh, sidx, didx, rows,
                  sisems, disems, gsems, ssems, S1, NS * S0 + s * S1)

    plsc.subcore_barrier()
    sl = pl.ds(s * SL, SL)
    pltpu.sync_copy(acc_sh.at[sl], out_hbm.at[c, sl])


def _score_body(a_hbm, b_hbm, srcf_hbm, dstf_hbm, out_hbm,
                src_v, dst_v, a_v, b_v, out_v):
    c = lax.axis_index("c")
    s = lax.axis_index("s")
    w = c * NS + s
    pltpu.sync_copy(srcf_hbm.at[pl.ds(w * ET, ET)], src_v)
    pltpu.sync_copy(dstf_hbm.at[pl.ds(w * ET, ET)], dst_v)
    pltpu.sync_copy(a_hbm, a_v)
    pltpu.sync_copy(b_hbm, b_v)

    def body(j, carry):
        ii = pl.ds(j * L, L)
        va = plsc.load_gather(a_v, [src_v[ii]])
        vb = plsc.load_gather(b_v, [dst_v[ii]])
        out_v[ii] = 1.0 / (1.0 + jnp.exp(-(va + vb)))
        return carry

    lax.fori_loop(0, ET // L, body, 0)
    pltpu.sync_copy(out_v, out_hbm.at[pl.ds(w * ET, ET)])


@functools.cache
def _sc_kernels():
    mesh = plsc.VectorSubcoreMesh(
        core_axis_name="c", subcore_axis_name="s",
        num_cores=NC, num_subcores=NS,
    )
    deg = pl.kernel(
        _deg_body,
        out_type=jax.ShapeDtypeStruct((NC, 2, NP, L), jnp.float32),
        mesh=mesh,
        compiler_params=pltpu.CompilerParams(use_tc_tiling_on_sc=False),
        scratch_types=[
            pltpu.VMEM((C, CH), jnp.int32),
            pltpu.VMEM((C, CH), jnp.int32),
            pltpu.VMEM((CH, L), jnp.float32),
            pltpu.VMEM_SHARED((NP, L), jnp.float32),
            pltpu.VMEM_SHARED((NP, L), jnp.float32),
            pltpu.SemaphoreType.DMA,
            pltpu.SemaphoreType.DMA,
        ],
    )
    spmm = pl.kernel(
        _spmm_body,
        out_type=jax.ShapeDtypeStruct((NC, NP, D), jnp.float32),
        mesh=mesh,
        compiler_params=pltpu.CompilerParams(use_tc_tiling_on_sc=False),
        scratch_types=[
            pltpu.VMEM((RI, CHS), jnp.int32),
            pltpu.VMEM((RI, CHS), jnp.int32),
            pltpu.VMEM((RR, CHS, D), jnp.float32),
            pltpu.VMEM_SHARED((NP, D), jnp.float32),
            pltpu.SemaphoreType.DMA((RI,)),
            pltpu.SemaphoreType.DMA((RI,)),
            pltpu.SemaphoreType.DMA((RR,)),
            pltpu.SemaphoreType.DMA((RR,)),
        ],
    )
    score = pl.kernel(
        _score_body,
        out_type=jax.ShapeDtypeStruct((EP,), jnp.float32),
        mesh=mesh,
        compiler_params=pltpu.CompilerParams(needs_layout_passes=False),
        scratch_types=[
            pltpu.VMEM((ET,), jnp.int32),
            pltpu.VMEM((ET,), jnp.int32),
            pltpu.VMEM((NP,), jnp.float32),
            pltpu.VMEM((NP,), jnp.float32),
            pltpu.VMEM((ET,), jnp.float32),
        ],
    )
    return deg, spmm, score


def _k1_body(deg_ref, x_ref, w1_ref, h1_ref, ns_ref, nd_ref):
    deg = deg_ref[...]
    ns = lax.rsqrt(jnp.clip(deg[0, 0] + deg[1, 0], 1.0, None))
    nd = lax.rsqrt(jnp.clip(deg[0, 1] + deg[1, 1], 1.0, None))
    ns_ref[...] = ns
    nd_ref[...] = nd
    xw = jnp.dot(x_ref[...], w1_ref[...], preferred_element_type=jnp.float32)
    h1_ref[...] = xw * ns[:, 0:1]


def _k1(deg, x_pad, w1):
    return pl.pallas_call(
        _k1_body,
        grid=(NB,),
        in_specs=[
            pl.BlockSpec((NC, 2, BLK, L), lambda i: (0, 0, i, 0)),
            pl.BlockSpec((BLK, D), lambda i: (i, 0)),
            pl.BlockSpec((D, D), lambda i: (0, 0)),
        ],
        out_specs=[
            pl.BlockSpec((BLK, D), lambda i: (i, 0)),
            pl.BlockSpec((BLK, L), lambda i: (i, 0)),
            pl.BlockSpec((BLK, L), lambda i: (i, 0)),
        ],
        out_shape=[
            jax.ShapeDtypeStruct((NP, D), jnp.float32),
            jax.ShapeDtypeStruct((NP, L), jnp.float32),
            jax.ShapeDtypeStruct((NP, L), jnp.float32),
        ],
    )(deg, x_pad, w1)


def _k2_body(agg_ref, ns_ref, nd_ref, b1_ref, w2_ref, h2_ref):
    agg = agg_ref[0] + agg_ref[1]
    x1 = jnp.maximum(agg * nd_ref[...][:, 0:1] + b1_ref[...], 0.0)
    h2_ref[...] = jnp.dot(x1 * ns_ref[...][:, 0:1], w2_ref[...],
                          preferred_element_type=jnp.float32)


def _k2(agg1, ns16, nd16, b1r, w2):
    return pl.pallas_call(
        _k2_body,
        grid=(NB,),
        in_specs=[
            pl.BlockSpec((NC, BLK, D), lambda i: (0, i, 0)),
            pl.BlockSpec((BLK, L), lambda i: (i, 0)),
            pl.BlockSpec((BLK, L), lambda i: (i, 0)),
            pl.BlockSpec((1, D), lambda i: (0, 0)),
            pl.BlockSpec((D, D), lambda i: (0, 0)),
        ],
        out_specs=pl.BlockSpec((BLK, D), lambda i: (i, 0)),
        out_shape=jax.ShapeDtypeStruct((NP, D), jnp.float32),
    )(agg1, ns16, nd16, b1r, w2)


def _k3_body(agg_ref, nd_ref, b2_ref, wp1_ref, wp2_ref, bp_ref, a_ref, bt_ref):
    agg = agg_ref[0] + agg_ref[1]
    x2 = jnp.maximum(agg * nd_ref[...][:, 0:1] + b2_ref[...], 0.0)
    a_ref[...] = jnp.sum(x2 * wp1_ref[...], axis=1) + bp_ref[0, 0]
    bt_ref[...] = jnp.sum(x2 * wp2_ref[...], axis=1)


def _k3(agg2, nd16, b2r, wp1, wp2, bpr):
    return pl.pallas_call(
        _k3_body,
        out_shape=[
            jax.ShapeDtypeStruct((NP,), jnp.float32),
            jax.ShapeDtypeStruct((NP,), jnp.float32),
        ],
    )(agg2, nd16, b2r, wp1, wp2, bpr)


def kernel(features, edge_index, edge_type, W1, b1, W2, b2, Wp, bp):
    src = edge_index[0]
    dst = edge_index[1]
    pad = jnp.full((EP - E,), N, dtype=jnp.int32)
    srcf = jnp.concatenate([src, pad])
    dstf = jnp.concatenate([dst, pad])
    src2d = srcf.reshape(R2, CH)
    dst2d = dstf.reshape(R2, CH)
    x_pad = jnp.pad(features, ((0, NP - N), (0, 0)))
    ones_cl = jnp.ones((CH, L), jnp.float32)
    zeros_l = jnp.zeros((SL, L), jnp.float32)
    zeros_d = jnp.zeros((SL, D), jnp.float32)

    src2ds = srcf.reshape(RS, CHS)
    dst2ds = dstf.reshape(RS, CHS)
    deg_kernel, spmm_kernel, score_kernel = _sc_kernels()
    deg = deg_kernel(src2d, dst2d, ones_cl, zeros_l)
    h1, ns16, nd16 = _k1(deg, x_pad, W1)
    agg1 = spmm_kernel(h1, src2ds, dst2ds, zeros_d)
    h2 = _k2(agg1, ns16, nd16, b1.reshape(1, D), W2)
    agg2 = spmm_kernel(h2, src2ds, dst2ds, zeros_d)
    a1d, b1d = _k3(agg2, nd16, b2.reshape(1, D),
                   Wp[:D, 0].reshape(1, D), Wp[D:, 0].reshape(1, D),
                   bp.reshape(1, 1))
    out = score_kernel(a1d, b1d, srcf, dstf)
    return out[:E]


# speedup vs baseline: 1.0392x; 1.0392x over previous
import functools

import jax
import jax.numpy as jnp
from jax import lax
from jax.experimental import pallas as pl
from jax.experimental.pallas import tpu as pltpu
from jax.experimental.pallas import tpu_sc as plsc

N = 10000
NP = 10240
D = 128
E = 320000
NC, NS, L = 2, 16, 16
NT = NC * NS
CH = 64
C = 160
ET = C * CH
EP = ET * NT
R2 = EP // CH
SL = NP // NS
BLK = 512
NB = NP // BLK


def _deg_body(src_hbm, dst_hbm, ones_hbm, zeros_hbm, out_hbm,
              src_v, dst_v, ones_v, degs_sh, degd_sh, sem_s, sem_d):
    c = lax.axis_index("c")
    s = lax.axis_index("s")
    w = c * NS + s
    pltpu.sync_copy(src_hbm.at[pl.ds(w * C, C)], src_v)
    pltpu.sync_copy(dst_hbm.at[pl.ds(w * C, C)], dst_v)
    pltpu.sync_copy(ones_hbm, ones_v)
    pltpu.sync_copy(zeros_hbm, degs_sh.at[pl.ds(s * SL, SL)])
    pltpu.sync_copy(zeros_hbm, degd_sh.at[pl.ds(s * SL, SL)])
    plsc.subcore_barrier()

    def body(jj, carry):
        for u in range(4):
            j = jj * 4 + u
            pltpu.async_copy(ones_v, degs_sh.at[src_v.at[j]], sem_s,
                             add=True)
            pltpu.async_copy(ones_v, degd_sh.at[dst_v.at[j]], sem_d,
                             add=True)
        for u in range(4):
            j = jj * 4 + u
            pltpu.make_async_copy(ones_v, degs_sh.at[src_v.at[j]],
                                  sem_s).wait()
            pltpu.make_async_copy(ones_v, degd_sh.at[dst_v.at[j]],
                                  sem_d).wait()
        return carry

    lax.fori_loop(0, C // 4, body, 0)
    plsc.subcore_barrier()
    sl = pl.ds(s * SL, SL)
    pltpu.sync_copy(degs_sh.at[sl], out_hbm.at[c, 0, sl])
    pltpu.sync_copy(degd_sh.at[sl], out_hbm.at[c, 1, sl])


CHS = 128
CT = EP // (CHS * NT)
S0 = 124
S1 = 2 * CT - S0
RS = EP // CHS
RI = 4
RR = 2


def _spmm_run(h_hbm, src_hbm, dst_hbm, acc_sh,
              sidx, didx, rows, sisems, disems, gsems, ssems, S, base):
    for b in range(RI):
        pltpu.async_copy(src_hbm.at[base + b], sidx.at[b], sisems.at[b])
        pltpu.async_copy(dst_hbm.at[base + b], didx.at[b], disems.at[b])

    def body(jj, carry):
        for b in range(RI):
            j = jj * RI + b
            r = b % RR
            b2 = (b + 2) % RI
            pltpu.make_async_copy(src_hbm.at[base + j], sidx.at[b],
                                  sisems.at[b]).wait()
            pltpu.make_async_copy(dst_hbm.at[base + j], didx.at[b],
                                  disems.at[b]).wait()

            def drain_and_prefetch():
                pltpu.make_async_copy(rows.at[r], acc_sh.at[didx.at[b2]],
                                      ssems.at[r]).wait()
                pltpu.async_copy(src_hbm.at[base + j + 2], sidx.at[b2],
                                 sisems.at[b2])
                pltpu.async_copy(dst_hbm.at[base + j + 2], didx.at[b2],
                                 disems.at[b2])

            if b >= 2:
                if b >= RI - 2:
                    @pl.when(jj < S // RI - 1)
                    def _():
                        drain_and_prefetch()

                    @pl.when(jj == S // RI - 1)
                    def _():
                        pltpu.make_async_copy(
                            rows.at[r], acc_sh.at[didx.at[b2]], ssems.at[r]
                        ).wait()
                else:
                    drain_and_prefetch()
            else:
                @pl.when(jj > 0)
                def _():
                    drain_and_prefetch()

            pltpu.async_copy(h_hbm.at[sidx.at[b]], rows.at[r], gsems.at[r])
            pltpu.make_async_copy(h_hbm.at[sidx.at[b]], rows.at[r],
                                  gsems.at[r]).wait()
            pltpu.async_copy(rows.at[r], acc_sh.at[didx.at[b]],
                             ssems.at[r], add=True)
        return carry

    lax.fori_loop(0, S // RI, body, 0)
    for r in range(RR):
        pltpu.make_async_copy(rows.at[r], acc_sh.at[didx.at[r]],
                              ssems.at[r]).wait()


def _spmm_body(h_hbm, src_hbm, dst_hbm, zeros_hbm, out_hbm,
               sidx, didx, rows, acc_sh, sisems, disems, gsems, ssems):
    c = lax.axis_index("c")
    s = lax.axis_index("s")
    pltpu.sync_copy(zeros_hbm, acc_sh.at[pl.ds(s * SL, SL)])
    plsc.subcore_barrier()

    @pl.when(c == 0)
    def _():
        _spmm_run(h_hbm, src_hbm, dst_hbm, acc_sh, sidx, didx, rows,
                  sisems, disems, gsems, ssems, S0, s * S0)

    @pl.when(c == 1)
    def _():
        _spmm_run(h_hbm, src_hbm, dst_hbm, acc_sh, sidx, didx, rows,
                  sisems, disems, gsems, ssems, S1, NS * S0 + s * S1)

    plsc.subcore_barrier()
    sl = pl.ds(s * SL, SL)
    pltpu.sync_copy(acc_sh.at[sl], out_hbm.at[c, sl])


def _score_body(a_hbm, b_hbm, srcf_hbm, dstf_hbm, out_hbm,
                src_v, dst_v, a_v, b_v, out_v):
    c = lax.axis_index("c")
    s = lax.axis_index("s")
    w = c * NS + s
    pltpu.sync_copy(srcf_hbm.at[pl.ds(w * ET, ET)], src_v)
    pltpu.sync_copy(dstf_hbm.at[pl.ds(w * ET, ET)], dst_v)
    pltpu.sync_copy(a_hbm, a_v)
    pltpu.sync_copy(b_hbm, b_v)

    def body(j, carry):
        ii = pl.ds(j * L, L)
        va = plsc.load_gather(a_v, [src_v[ii]])
        vb = plsc.load_gather(b_v, [dst_v[ii]])
        out_v[ii] = 1.0 / (1.0 + jnp.exp(-(va + vb)))
        return carry

    lax.fori_loop(0, ET // L, body, 0)
    pltpu.sync_copy(out_v, out_hbm.at[pl.ds(w * ET, ET)])


@functools.cache
def _sc_kernels():
    mesh = plsc.VectorSubcoreMesh(
        core_axis_name="c", subcore_axis_name="s",
        num_cores=NC, num_subcores=NS,
    )
    deg = pl.kernel(
        _deg_body,
        out_type=jax.ShapeDtypeStruct((NC, 2, NP, L), jnp.float32),
        mesh=mesh,
        compiler_params=pltpu.CompilerParams(use_tc_tiling_on_sc=False),
        scratch_types=[
            pltpu.VMEM((C, CH), jnp.int32),
            pltpu.VMEM((C, CH), jnp.int32),
            pltpu.VMEM((CH, L), jnp.float32),
            pltpu.VMEM_SHARED((NP, L), jnp.float32),
            pltpu.VMEM_SHARED((NP, L), jnp.float32),
            pltpu.SemaphoreType.DMA,
            pltpu.SemaphoreType.DMA,
        ],
    )
    spmm = pl.kernel(
        _spmm_body,
        out_type=jax.ShapeDtypeStruct((NC, NP, D), jnp.float32),
        mesh=mesh,
        compiler_params=pltpu.CompilerParams(use_tc_tiling_on_sc=False),
        scratch_types=[
            pltpu.VMEM((RI, CHS), jnp.int32),
            pltpu.VMEM((RI, CHS), jnp.int32),
            pltpu.VMEM((RR, CHS, D), jnp.float32),
            pltpu.VMEM_SHARED((NP, D), jnp.float32),
            pltpu.SemaphoreType.DMA((RI,)),
            pltpu.SemaphoreType.DMA((RI,)),
            pltpu.SemaphoreType.DMA((RR,)),
            pltpu.SemaphoreType.DMA((RR,)),
        ],
    )
    score = pl.kernel(
        _score_body,
        out_type=jax.ShapeDtypeStruct((EP,), jnp.float32),
        mesh=mesh,
        compiler_params=pltpu.CompilerParams(needs_layout_passes=False),
        scratch_types=[
            pltpu.VMEM((ET,), jnp.int32),
            pltpu.VMEM((ET,), jnp.int32),
            pltpu.VMEM((NP,), jnp.float32),
            pltpu.VMEM((NP,), jnp.float32),
            pltpu.VMEM((ET,), jnp.float32),
        ],
    )
    return deg, spmm, score


def _k1_body(deg_ref, x_ref, w1_ref, h1_ref, ns_ref, nd_ref):
    deg = deg_ref[...]
    ns = lax.rsqrt(jnp.clip(deg[0, 0] + deg[1, 0], 1.0, None))
    nd = lax.rsqrt(jnp.clip(deg[0, 1] + deg[1, 1], 1.0, None))
    ns_ref[...] = ns
    nd_ref[...] = nd
    xw = jnp.dot(x_ref[...], w1_ref[...], preferred_element_type=jnp.float32)
    h1_ref[...] = xw * ns[:, 0:1]


def _k1(deg, x_pad, w1):
    return pl.pallas_call(
        _k1_body,
        grid=(NB,),
        in_specs=[
            pl.BlockSpec((NC, 2, BLK, L), lambda i: (0, 0, i, 0)),
            pl.BlockSpec((BLK, D), lambda i: (i, 0)),
            pl.BlockSpec((D, D), lambda i: (0, 0)),
        ],
        out_specs=[
            pl.BlockSpec((BLK, D), lambda i: (i, 0)),
            pl.BlockSpec((BLK, L), lambda i: (i, 0)),
            pl.BlockSpec((BLK, L), lambda i: (i, 0)),
        ],
        out_shape=[
            jax.ShapeDtypeStruct((NP, D), jnp.float32),
            jax.ShapeDtypeStruct((NP, L), jnp.float32),
            jax.ShapeDtypeStruct((NP, L), jnp.float32),
        ],
    )(deg, x_pad, w1)


def _k2_body(agg_ref, ns_ref, nd_ref, b1_ref, w2_ref, h2_ref):
    agg = agg_ref[0] + agg_ref[1]
    x1 = jnp.maximum(agg * nd_ref[...][:, 0:1] + b1_ref[...], 0.0)
    h2_ref[...] = jnp.dot(x1 * ns_ref[...][:, 0:1], w2_ref[...],
                          preferred_element_type=jnp.float32)


def _k2(agg1, ns16, nd16, b1r, w2):
    return pl.pallas_call(
        _k2_body,
        grid=(NB,),
        in_specs=[
            pl.BlockSpec((NC, BLK, D), lambda i: (0, i, 0)),
            pl.BlockSpec((BLK, L), lambda i: (i, 0)),
            pl.BlockSpec((BLK, L), lambda i: (i, 0)),
            pl.BlockSpec((1, D), lambda i: (0, 0)),
            pl.BlockSpec((D, D), lambda i: (0, 0)),
        ],
        out_specs=pl.BlockSpec((BLK, D), lambda i: (i, 0)),
        out_shape=jax.ShapeDtypeStruct((NP, D), jnp.float32),
    )(agg1, ns16, nd16, b1r, w2)


def _k3_body(agg_ref, nd_ref, b2_ref, wp1_ref, wp2_ref, bp_ref, a_ref, bt_ref):
    agg = agg_ref[0] + agg_ref[1]
    x2 = jnp.maximum(agg * nd_ref[...][:, 0:1] + b2_ref[...], 0.0)
    a_ref[...] = jnp.sum(x2 * wp1_ref[...], axis=1) + bp_ref[0, 0]
    bt_ref[...] = jnp.sum(x2 * wp2_ref[...], axis=1)


def _k3(agg2, nd16, b2r, wp1, wp2, bpr):
    return pl.pallas_call(
        _k3_body,
        out_shape=[
            jax.ShapeDtypeStruct((NP,), jnp.float32),
            jax.ShapeDtypeStruct((NP,), jnp.float32),
        ],
    )(agg2, nd16, b2r, wp1, wp2, bpr)


def kernel(features, edge_index, edge_type, W1, b1, W2, b2, Wp, bp):
    src = edge_index[0]
    dst = edge_index[1]
    pad = jnp.full((EP - E,), N, dtype=jnp.int32)
    srcf = jnp.concatenate([src, pad])
    dstf = jnp.concatenate([dst, pad])
    src2d = srcf.reshape(R2, CH)
    dst2d = dstf.reshape(R2, CH)
    x_pad = jnp.pad(features, ((0, NP - N), (0, 0)))
    ones_cl = jnp.ones((CH, L), jnp.float32)
    zeros_l = jnp.zeros((SL, L), jnp.float32)
    zeros_d = jnp.zeros((SL, D), jnp.float32)

    src2ds = srcf.reshape(RS, CHS)
    dst2ds = dstf.reshape(RS, CHS)
    deg_kernel, spmm_kernel, score_kernel = _sc_kernels()
    deg = deg_kernel(src2d, dst2d, ones_cl, zeros_l)
    h1, ns16, nd16 = _k1(deg, x_pad, W1)
    agg1 = spmm_kernel(h1, src2ds, dst2ds, zeros_d)
    h2 = _k2(agg1, ns16, nd16, b1.reshape(1, D), W2)
    agg2 = spmm_kernel(h2, src2ds, dst2ds, zeros_d)
    a1d, b1d = _k3(agg2, nd16, b2.reshape(1, D),
                   Wp[:D, 0].reshape(1, D), Wp[D:, 0].reshape(1, D),
                   bp.reshape(1, 1))
    out = score_kernel(a1d, b1d, srcf, dstf)
    return out[:E]
